# Initial kernel scaffold; baseline (speedup 1.0000x reference)
import functools

import jax
import jax.numpy as jnp
from jax import lax
from jax.experimental import pallas as pl
from jax.experimental.pallas import tpu as pltpu
from jax.experimental.pallas import tpu_sc as plsc

B = 16384
D = 64
H = 128
K = 6
NC = 2
NS = 16
NW = NC * NS
BPW = B // NW
CH = 128
NCH = BPW // CH


def _sc_gather(idx, t0, t1, t2, t3, t4, t5):
  mesh = plsc.VectorSubcoreMesh(core_axis_name="c", subcore_axis_name="s")

  @functools.partial(
      pl.kernel,
      out_type=jax.ShapeDtypeStruct((K, NW, NCH, CH, D), jnp.float32),
      mesh=mesh,
      scratch_types=[
          pltpu.VMEM((NCH, CH), jnp.int32),
          pltpu.VMEM((NCH, CH, D), jnp.float32),
          pltpu.SemaphoreType.DMA,
      ],
  )
  def body(idx_hbm, a0, a1, a2, a3, a4, a5, out_hbm, idx_v, rows_v, sem):
    wid = lax.axis_index("s") * NC + lax.axis_index("c")
    tabs = (a0, a1, a2, a3, a4, a5)
    for k in range(K):
      pltpu.sync_copy(idx_hbm.at[k, wid], idx_v)
      handles = []
      for j in range(NCH):
        handles.append(
            pltpu.async_copy(tabs[k].at[idx_v.at[j]], rows_v.at[j], sem))
      for h in handles:
        h.wait()
      pltpu.sync_copy(rows_v, out_hbm.at[k, wid])

  return body(idx, t0, t1, t2, t3, t4, t5)


def _tc_project(g, w, bias):
  bm = 1024

  def body(g_ref, w_ref, b_ref, o_ref):
    acc = jnp.broadcast_to(b_ref[0], (bm, H))
    for k in range(K):
      acc = acc + jnp.dot(g_ref[k], w_ref[k],
                          preferred_element_type=jnp.float32)
    o_ref[...] = acc

  return pl.pallas_call(
      body,
      grid=(B // bm,),
      in_specs=[
          pl.BlockSpec((K, bm, D), lambda i: (0, i, 0)),
          pl.BlockSpec((K, D, H), lambda i: (0, 0, 0)),
          pl.BlockSpec((1, H), lambda i: (0, 0)),
      ],
      out_specs=pl.BlockSpec((bm, H), lambda i: (i, 0)),
      out_shape=jax.ShapeDtypeStruct((B, H), jnp.float32),
  )(g, w, bias)


def kernel(batch_seq_cat, t_incbd, t_bldcnt, t_floorn, t_area, t_lon, t_lat,
           W, b):
  idx = batch_seq_cat[:, 1:7].astype(jnp.int32).T.reshape(K, NW, NCH, CH)
  g = _sc_gather(idx, t_incbd, t_bldcnt, t_floorn, t_area, t_lon, t_lat)
  g = g.reshape(K, B, D)
  w = W.reshape(K, D, H)
  bias = b.reshape(1, H)
  return _tc_project(g, w, bias)


# speedup vs baseline: 1.5195x; 1.5195x over previous
import functools

import jax
import jax.numpy as jnp
from jax import lax
from jax.experimental import pallas as pl
from jax.experimental.pallas import tpu as pltpu
from jax.experimental.pallas import tpu_sc as plsc

B = 16384
D = 64
H = 128
K = 6
NC = 2
NS = 16
NW = NC * NS
BPW = B // NW
CH = 128
NCH = BPW // CH


def _sc_gather(idx, t0, t1, t2, t3, t4, t5):
  mesh = plsc.VectorSubcoreMesh(core_axis_name="c", subcore_axis_name="s")

  @functools.partial(
      pl.kernel,
      out_type=jax.ShapeDtypeStruct((K, NW, NCH, CH, D), jnp.float32),
      mesh=mesh,
      scratch_types=[
          pltpu.VMEM((NCH, CH), jnp.int32),
          pltpu.VMEM((NCH, CH, D), jnp.float32),
          pltpu.SemaphoreType.DMA,
      ],
      compiler_params=pltpu.CompilerParams(use_tc_tiling_on_sc=False),
  )
  def body(idx_hbm, a0, a1, a2, a3, a4, a5, out_hbm, idx_v, rows_v, sem):
    wid = lax.axis_index("s") * NC + lax.axis_index("c")
    tabs = (a0, a1, a2, a3, a4, a5)
    for k in range(K):
      pltpu.sync_copy(idx_hbm.at[k, wid], idx_v)
      handles = []
      for j in range(NCH):
        handles.append(
            pltpu.async_copy(tabs[k].at[idx_v.at[j]], rows_v.at[j], sem))
      for h in handles:
        h.wait()
      pltpu.sync_copy(rows_v, out_hbm.at[k, wid])

  return body(idx, t0, t1, t2, t3, t4, t5)


def _tc_project(g, w, bias):
  bm = 1024

  def body(g_ref, w_ref, b_ref, o_ref):
    acc = jnp.broadcast_to(b_ref[0], (bm, H))
    for k in range(K):
      acc = acc + jnp.dot(g_ref[k], w_ref[k],
                          preferred_element_type=jnp.float32)
    o_ref[...] = acc

  return pl.pallas_call(
      body,
      grid=(B // bm,),
      in_specs=[
          pl.BlockSpec((K, bm, D), lambda i: (0, i, 0)),
          pl.BlockSpec((K, D, H), lambda i: (0, 0, 0)),
          pl.BlockSpec((1, H), lambda i: (0, 0)),
      ],
      out_specs=pl.BlockSpec((bm, H), lambda i: (i, 0)),
      out_shape=jax.ShapeDtypeStruct((B, H), jnp.float32),
  )(g, w, bias)


def kernel(batch_seq_cat, t_incbd, t_bldcnt, t_floorn, t_area, t_lon, t_lat,
           W, b):
  idx = batch_seq_cat[:, 1:7].astype(jnp.int32).T.reshape(K, NW, NCH, CH)
  g = _sc_gather(idx, t_incbd, t_bldcnt, t_floorn, t_area, t_lon, t_lat)
  g = g.reshape(K, B, D)
  w = W.reshape(K, D, H)
  bias = b.reshape(1, H)
  return _tc_project(g, w, bias)


# speedup vs baseline: 1.7047x; 1.1218x over previous
import functools

import jax
import jax.numpy as jnp
from jax import lax
from jax.experimental import pallas as pl
from jax.experimental.pallas import tpu as pltpu
from jax.experimental.pallas import tpu_sc as plsc

B = 16384
D = 64
H = 128
K = 6
NC = 2
NS = 16
NW = NC * NS
BPW = B // NW
CH = 128
NCH = BPW // CH


def _sc_gather(idx, t0, t1, t2, t3, t4, t5):
  mesh = plsc.VectorSubcoreMesh(core_axis_name="c", subcore_axis_name="s")

  @functools.partial(
      pl.kernel,
      out_type=jax.ShapeDtypeStruct((K // 2, NW, NCH, CH, 2 * D),
                                    jnp.float32),

      mesh=mesh,
      scratch_types=[
          pltpu.VMEM((2, NCH, CH), jnp.int32),
          pltpu.VMEM((2, NCH, CH, D), jnp.float32),
          pltpu.SemaphoreType.DMA,
      ],
      compiler_params=pltpu.CompilerParams(use_tc_tiling_on_sc=False),
  )
  def body(idx_hbm, a0, a1, a2, a3, a4, a5, out_hbm, idx_v, rows_v, sem):
    wid = lax.axis_index("s") * NC + lax.axis_index("c")
    tabs = (a0, a1, a2, a3, a4, a5)
    for p in range(K // 2):
      pltpu.sync_copy(idx_hbm.at[2 * p, wid], idx_v.at[0])
      pltpu.sync_copy(idx_hbm.at[2 * p + 1, wid], idx_v.at[1])
      handles = []
      for h in range(2):
        for j in range(NCH):
          handles.append(
              pltpu.async_copy(
                  tabs[2 * p + h].at[idx_v.at[h, j]],
                  rows_v.at[h, j],
                  sem,
              ))
      for hd in handles:
        hd.wait()
      for h in range(2):
        pltpu.sync_copy(
            rows_v.at[h],
            out_hbm.at[p, wid, slice(None), slice(None), pl.ds(h * D, D)])

  return body(idx, t0, t1, t2, t3, t4, t5)


def _tc_project(g, w, bias):
  bm = 1024

  def body(g_ref, w_ref, b_ref, o_ref):
    acc = jnp.broadcast_to(b_ref[0], (bm, H))
    for p in range(K // 2):
      acc = acc + jnp.dot(g_ref[p], w_ref[p],
                          preferred_element_type=jnp.float32)
    o_ref[...] = acc

  return pl.pallas_call(
      body,
      grid=(B // bm,),
      in_specs=[
          pl.BlockSpec((K // 2, bm, 2 * D), lambda i: (0, i, 0)),
          pl.BlockSpec((K // 2, 2 * D, H), lambda i: (0, 0, 0)),
          pl.BlockSpec((1, H), lambda i: (0, 0)),
      ],
      out_specs=pl.BlockSpec((bm, H), lambda i: (i, 0)),
      out_shape=jax.ShapeDtypeStruct((B, H), jnp.float32),
  )(g, w, bias)


def kernel(batch_seq_cat, t_incbd, t_bldcnt, t_floorn, t_area, t_lon, t_lat,
           W, b):
  idx = batch_seq_cat[:, 1:7].astype(jnp.int32).T.reshape(K, NW, NCH, CH)
  g = _sc_gather(idx, t_incbd, t_bldcnt, t_floorn, t_area, t_lon, t_lat)
  g = g.reshape(K // 2, B, 2 * D)
  w = W.reshape(K // 2, 2 * D, H)
  bias = b.reshape(1, H)
  return _tc_project(g, w, bias)


# speedup vs baseline: 2.3241x; 1.3634x over previous
import functools

import jax
import jax.numpy as jnp
from jax import lax
from jax.experimental import pallas as pl
from jax.experimental.pallas import tpu as pltpu
from jax.experimental.pallas import tpu_sc as plsc

B = 16384
V = 100000
D = 64
H = 128
K = 6
G = 3
NC = 2
NS = 16
NW = NC * NS
BPW = B // NW
CH = 128
NCH = BPW // CH

TB = 512
M = 50176
NTB = M // TB


def _tc_pack(*tabs_t):
  def body(*refs):
    ins, outs = refs[:2 * G], refs[2 * G:]
    for k in range(G):
      xc = jnp.concatenate([ins[2 * k][...], ins[2 * k + 1][...]], axis=0)
      outs[k][...] = xc.T

  in_specs = []
  for _ in range(G):
    in_specs.append(pl.BlockSpec((D, TB), lambda i: (0, i)))
    in_specs.append(pl.BlockSpec((D, TB), lambda i: (0, i + NTB)))
  return pl.pallas_call(
      body,
      grid=(NTB,),
      in_specs=in_specs,
      out_specs=[pl.BlockSpec((TB, 2 * D), lambda i: (i, 0))] * G,
      out_shape=[jax.ShapeDtypeStruct((M, 2 * D), jnp.float32)] * G,
  )(*[t for t in tabs_t for _ in range(2)])


def _sc_gather(idxp, *tabs):
  mesh = plsc.VectorSubcoreMesh(core_axis_name="c", subcore_axis_name="s")

  @functools.partial(
      pl.kernel,
      out_type=jax.ShapeDtypeStruct((G, NW, NCH, CH, 2 * D), jnp.float32),
      mesh=mesh,
      scratch_types=[
          pltpu.VMEM((8, CH), jnp.int32),
          pltpu.VMEM((NCH, CH, 2 * D), jnp.float32),
          pltpu.SemaphoreType.DMA,
      ],
  )
  def body(idx_hbm, a0, a1, a2, out_hbm, idx_v, rows_v, sem):
    wid = lax.axis_index("s") * NC + lax.axis_index("c")
    tabs_r = (a0, a1, a2)
    for k in range(G):
      pltpu.sync_copy(idx_hbm.at[k, wid], idx_v)
      handles = []
      for j in range(NCH):
        handles.append(
            pltpu.async_copy(tabs_r[k].at[idx_v.at[j]], rows_v.at[j], sem))
      for hd in handles:
        hd.wait()
      pltpu.sync_copy(rows_v, out_hbm.at[k, wid])

  return body(idxp, *tabs)


def _tc_project(g0, g1, half, w, bias):
  bm = 1024

  def body(g0_ref, g1_ref, p_ref, w_ref, b_ref, o_ref):
    acc = jnp.broadcast_to(b_ref[0], (bm, H))
    for k in range(K):
      g_ref = g0_ref if k < G else g1_ref
      gk = g_ref[k % G]
      pk = p_ref[k][:, None]
      ek = gk[:, :D] * (1.0 - pk) + gk[:, D:] * pk
      acc = acc + jnp.dot(ek, w_ref[k], preferred_element_type=jnp.float32)
    o_ref[...] = acc

  return pl.pallas_call(
      body,
      grid=(B // bm,),
      in_specs=[
          pl.BlockSpec((G, bm, 2 * D), lambda i: (0, i, 0)),
          pl.BlockSpec((G, bm, 2 * D), lambda i: (0, i, 0)),
          pl.BlockSpec((8, bm), lambda i: (0, i)),
          pl.BlockSpec((K, D, H), lambda i: (0, 0, 0)),
          pl.BlockSpec((1, H), lambda i: (0, 0)),
      ],
      out_specs=pl.BlockSpec((bm, H), lambda i: (i, 0)),
      out_shape=jax.ShapeDtypeStruct((B, H), jnp.float32),
  )(g0, g1, half, w, bias)


def kernel(batch_seq_cat, t_incbd, t_bldcnt, t_floorn, t_area, t_lon, t_lat,
           W, b):
  idx = batch_seq_cat[:, 1:7].astype(jnp.int32).T
  hbit = (idx >= M).astype(jnp.int32)
  idxp = (idx - hbit * M).reshape(K, NW, NCH, CH)
  idxp = jnp.pad(idxp, ((0, 0), (0, 0), (0, 8 - NCH), (0, 0)))
  half = jnp.pad(hbit.astype(jnp.float32), ((0, 2), (0, 0)))
  packed0 = _tc_pack(t_incbd.T, t_bldcnt.T, t_floorn.T)
  packed1 = _tc_pack(t_area.T, t_lon.T, t_lat.T)
  g0 = _sc_gather(idxp[:G], *packed0)
  g1 = _sc_gather(idxp[G:], *packed1)
  g0 = g0.reshape(G, B, 2 * D)
  g1 = g1.reshape(G, B, 2 * D)
  w = W.reshape(K, D, H)
  bias = b.reshape(1, H)
  return _tc_project(g0, g1, half, w, bias)


# speedup vs baseline: 2.8716x; 1.2356x over previous
import functools

import jax
import jax.numpy as jnp
from jax import lax
from jax.experimental import pallas as pl
from jax.experimental.pallas import tpu as pltpu
from jax.experimental.pallas import tpu_sc as plsc

B = 16384
V = 100000
D = 64
H = 128
K = 6
NC = 2
NS = 16
NW = NC * NS
BPW = B // NW
CH = 128
NCH = BPW // CH
NCHUNK = K * NCH
SLOTS = 4

TB = 512
M = 50176
NTB = M // TB


def _tc_pack(*tabs_t):
  def body(*refs):
    ins, outs = refs[:2 * K], refs[2 * K:]
    for k in range(K):
      xc = jnp.concatenate([ins[2 * k][...], ins[2 * k + 1][...]], axis=0)
      outs[k][...] = xc.T

  in_specs = []
  for _ in range(K):
    in_specs.append(pl.BlockSpec((D, TB), lambda i: (0, i)))
    in_specs.append(pl.BlockSpec((D, TB), lambda i: (0, i + NTB)))
  return pl.pallas_call(
      body,
      grid=(NTB,),
      in_specs=in_specs,
      out_specs=[pl.BlockSpec((TB, 2 * D), lambda i: (i, 0))] * K,
      out_shape=[jax.ShapeDtypeStruct((M, 2 * D), jnp.float32)] * K,
  )(*[t for t in tabs_t for _ in range(2)])


def _sc_gather(idxp, *tabs):
  mesh = plsc.VectorSubcoreMesh(core_axis_name="c", subcore_axis_name="s")

  @functools.partial(
      pl.kernel,
      out_type=jax.ShapeDtypeStruct((K, NW, NCH, CH, 2 * D), jnp.float32),
      mesh=mesh,
      scratch_types=[
          pltpu.VMEM((NCHUNK, CH), jnp.int32),
          pltpu.VMEM((SLOTS, CH, 2 * D), jnp.float32),
          pltpu.SemaphoreType.DMA,
          pltpu.SemaphoreType.DMA,
          pltpu.SemaphoreType.DMA,
          pltpu.SemaphoreType.DMA,
      ],
  )
  def body(idx_hbm, a0, a1, a2, a3, a4, a5, out_hbm, idx_v, rows_v, gsem0,
           gsem1, wsem0, wsem1):
    wid = lax.axis_index("s") * NC + lax.axis_index("c")
    tabs_r = (a0, a1, a2, a3, a4, a5)
    gsems = (gsem0, gsem1)
    wsems = (wsem0, wsem1)
    pltpu.sync_copy(idx_hbm.at[wid], idx_v)

    def fire(c):
      k = c // NCH
      pltpu.async_copy(
          tabs_r[k].at[idx_v.at[c]], rows_v.at[c % SLOTS], gsems[c % 2])

    def gwait(c):
      k = c // NCH
      pltpu.make_async_copy(
          tabs_r[k].at[idx_v.at[c]], rows_v.at[c % SLOTS],
          gsems[c % 2]).wait()

    def wstart(c):
      k, j = divmod(c, NCH)
      pltpu.async_copy(
          rows_v.at[c % SLOTS], out_hbm.at[k, wid, j], wsems[c % 2])

    def wwait(c):
      k, j = divmod(c, NCH)
      pltpu.make_async_copy(
          rows_v.at[c % SLOTS], out_hbm.at[k, wid, j],
          wsems[c % 2]).wait()

    fire(0)
    fire(1)
    for c in range(NCHUNK):
      gwait(c)
      if c >= 2:
        wwait(c - 2)
      wstart(c)
      if c + 2 < NCHUNK:
        fire(c + 2)
    for c in range(NCHUNK - 2, NCHUNK):
      wwait(c)

  return body(idxp, *tabs)


def _tc_project(g, half, w, bias):
  bm = 1024

  def body(g_ref, p_ref, w_ref, b_ref, o_ref):
    acc = jnp.broadcast_to(b_ref[0], (bm, H))
    for k in range(K):
      gk = g_ref[k]
      pk = p_ref[k][:, None]
      ek = gk[:, :D] * (1.0 - pk) + gk[:, D:] * pk
      acc = acc + jnp.dot(ek, w_ref[k], preferred_element_type=jnp.float32)
    o_ref[...] = acc

  return pl.pallas_call(
      body,
      grid=(B // bm,),
      in_specs=[
          pl.BlockSpec((K, bm, 2 * D), lambda i: (0, i, 0)),
          pl.BlockSpec((8, bm), lambda i: (0, i)),
          pl.BlockSpec((K, D, H), lambda i: (0, 0, 0)),
          pl.BlockSpec((1, H), lambda i: (0, 0)),
      ],
      out_specs=pl.BlockSpec((bm, H), lambda i: (i, 0)),
      out_shape=jax.ShapeDtypeStruct((B, H), jnp.float32),
  )(g, half, w, bias)


def kernel(batch_seq_cat, t_incbd, t_bldcnt, t_floorn, t_area, t_lon, t_lat,
           W, b):
  idx = batch_seq_cat[:, 1:7].astype(jnp.int32).T
  hbit = (idx >= M).astype(jnp.int32)
  idxp = (idx - hbit * M).reshape(K, NW, NCH, CH).transpose(1, 0, 2, 3)
  idxp = idxp.reshape(NW, NCHUNK, CH)
  half = jnp.pad(hbit.astype(jnp.float32), ((0, 2), (0, 0)))
  packed = _tc_pack(t_incbd.T, t_bldcnt.T, t_floorn.T, t_area.T, t_lon.T,
                    t_lat.T)
  g = _sc_gather(idxp, *packed)
  g = g.reshape(K, B, 2 * D)
  w = W.reshape(K, D, H)
  bias = b.reshape(1, H)
  return _tc_project(g, half, w, bias)


# speedup vs baseline: 3.2263x; 1.1235x over previous
import functools

import jax
import jax.numpy as jnp
from jax import lax
from jax.experimental import pallas as pl
from jax.experimental.pallas import tpu as pltpu
from jax.experimental.pallas import tpu_sc as plsc

B = 16384
V = 100000
D = 64
H = 128
K = 6
NC = 2
NS = 16
NW = NC * NS
BPW = B // NW
CH = 128
NCH = BPW // CH
NCHUNK = K * NCH
SLOTS = 4

TB = 1024
M = 50176
NTB = M // TB


def _tc_pack(*tabs_t):
  def body(*refs):
    ins, outs = refs[:2 * K], refs[2 * K:]
    for k in range(K):
      xc = jnp.concatenate([ins[2 * k][...], ins[2 * k + 1][...]], axis=0)
      outs[k][...] = xc.T

  in_specs = []
  for _ in range(K):
    in_specs.append(pl.BlockSpec((D, TB), lambda i: (0, i)))
    in_specs.append(pl.BlockSpec((D, TB), lambda i: (0, i + NTB)))
  return pl.pallas_call(
      body,
      grid=(NTB,),
      in_specs=in_specs,
      out_specs=[pl.BlockSpec((TB, 2 * D), lambda i: (i, 0))] * K,
      out_shape=[jax.ShapeDtypeStruct((M, 2 * D), jnp.float32)] * K,
  )(*[t for t in tabs_t for _ in range(2)])


def _sc_gather(idxp, *tabs):
  mesh = plsc.VectorSubcoreMesh(core_axis_name="c", subcore_axis_name="s")

  @functools.partial(
      pl.kernel,
      out_type=jax.ShapeDtypeStruct((K, NW, NCH, CH, 2 * D), jnp.float32),
      mesh=mesh,
      scratch_types=[
          pltpu.VMEM((NCHUNK, CH), jnp.int32),
          pltpu.VMEM((SLOTS, CH, 2 * D), jnp.float32),
          pltpu.SemaphoreType.DMA,
          pltpu.SemaphoreType.DMA,
          pltpu.SemaphoreType.DMA,
          pltpu.SemaphoreType.DMA,
      ],
  )
  def body(idx_hbm, a0, a1, a2, a3, a4, a5, out_hbm, idx_v, rows_v, gsem0,
           gsem1, wsem0, wsem1):
    wid = lax.axis_index("s") * NC + lax.axis_index("c")
    tabs_r = (a0, a1, a2, a3, a4, a5)
    gsems = (gsem0, gsem1)
    wsems = (wsem0, wsem1)
    pltpu.sync_copy(idx_hbm.at[wid], idx_v)

    def fire(c):
      k = c // NCH
      pltpu.async_copy(
          tabs_r[k].at[idx_v.at[c]], rows_v.at[c % SLOTS], gsems[c % 2])

    def gwait(c):
      k = c // NCH
      pltpu.make_async_copy(
          tabs_r[k].at[idx_v.at[c]], rows_v.at[c % SLOTS],
          gsems[c % 2]).wait()

    def wstart(c):
      k, j = divmod(c, NCH)
      pltpu.async_copy(
          rows_v.at[c % SLOTS], out_hbm.at[k, wid, j], wsems[c % 2])

    def wwait(c):
      k, j = divmod(c, NCH)
      pltpu.make_async_copy(
          rows_v.at[c % SLOTS], out_hbm.at[k, wid, j],
          wsems[c % 2]).wait()

    fire(0)
    fire(1)
    for c in range(NCHUNK):
      gwait(c)
      if c >= 2:
        wwait(c - 2)
      wstart(c)
      if c + 2 < NCHUNK:
        fire(c + 2)
    for c in range(NCHUNK - 2, NCHUNK):
      wwait(c)

  return body(idxp, *tabs)


def _tc_project(g, half, w, bias):
  bm = 1024

  def body(g_ref, p_ref, w_ref, b_ref, o_ref):
    acc = jnp.broadcast_to(b_ref[0], (bm, H))
    for k in range(K):
      gk = g_ref[k]
      pk = p_ref[k][:, None]
      ek = gk[:, :D] * (1.0 - pk) + gk[:, D:] * pk
      acc = acc + jnp.dot(ek, w_ref[k], preferred_element_type=jnp.float32)
    o_ref[...] = acc

  return pl.pallas_call(
      body,
      grid=(B // bm,),
      in_specs=[
          pl.BlockSpec((K, bm, 2 * D), lambda i: (0, i, 0)),
          pl.BlockSpec((8, bm), lambda i: (0, i)),
          pl.BlockSpec((K, D, H), lambda i: (0, 0, 0)),
          pl.BlockSpec((1, H), lambda i: (0, 0)),
      ],
      out_specs=pl.BlockSpec((bm, H), lambda i: (i, 0)),
      out_shape=jax.ShapeDtypeStruct((B, H), jnp.float32),
  )(g, half, w, bias)


def kernel(batch_seq_cat, t_incbd, t_bldcnt, t_floorn, t_area, t_lon, t_lat,
           W, b):
  idx = batch_seq_cat[:, 1:7].astype(jnp.int32).T
  hbit = (idx >= M).astype(jnp.int32)
  idxp = (idx - hbit * M).reshape(K, NW, NCH, CH).transpose(1, 0, 2, 3)
  idxp = idxp.reshape(NW, NCHUNK, CH)
  half = jnp.pad(hbit.astype(jnp.float32), ((0, 2), (0, 0)))
  packed = _tc_pack(t_incbd.T, t_bldcnt.T, t_floorn.T, t_area.T, t_lon.T,
                    t_lat.T)
  g = _sc_gather(idxp, *packed)
  g = g.reshape(K, B, 2 * D)
  w = W.reshape(K, D, H)
  bias = b.reshape(1, H)
  return _tc_project(g, half, w, bias)


# speedup vs baseline: 3.3679x; 1.0439x over previous
import functools

import jax
import jax.numpy as jnp
from jax import lax
from jax.experimental import pallas as pl
from jax.experimental.pallas import tpu as pltpu
from jax.experimental.pallas import tpu_sc as plsc

B = 16384
V = 100000
D = 64
H = 128
K = 6
NC = 2
NS = 16
NW = NC * NS
BPW = B // NW
CH = 128
NCH = BPW // CH
NCHUNK = K * NCH
SLOTS = 4

TB = 1792
M = 50176
NTB = M // TB


def _tc_pack(*tabs_t):
  def body(*refs):
    ins, outs = refs[:2 * K], refs[2 * K:]
    for k in range(K):
      xc = jnp.concatenate([ins[2 * k][...], ins[2 * k + 1][...]], axis=0)
      outs[k][...] = xc.T

  in_specs = []
  for _ in range(K):
    in_specs.append(pl.BlockSpec((D, TB), lambda i: (0, i)))
    in_specs.append(pl.BlockSpec((D, TB), lambda i: (0, i + NTB)))
  return pl.pallas_call(
      body,
      grid=(NTB,),
      in_specs=in_specs,
      out_specs=[pl.BlockSpec((TB, 2 * D), lambda i: (i, 0))] * K,
      out_shape=[jax.ShapeDtypeStruct((M, 2 * D), jnp.float32)] * K,
  )(*[t for t in tabs_t for _ in range(2)])


def _sc_gather(idxp, *tabs):
  mesh = plsc.VectorSubcoreMesh(core_axis_name="c", subcore_axis_name="s")

  @functools.partial(
      pl.kernel,
      out_type=jax.ShapeDtypeStruct((K, NW, NCH, CH, 2 * D), jnp.float32),
      mesh=mesh,
      scratch_types=[
          pltpu.VMEM((NCHUNK, CH), jnp.int32),
          pltpu.VMEM((SLOTS, CH, 2 * D), jnp.float32),
          pltpu.SemaphoreType.DMA,
          pltpu.SemaphoreType.DMA,
          pltpu.SemaphoreType.DMA,
          pltpu.SemaphoreType.DMA,
      ],
  )
  def body(idx_hbm, a0, a1, a2, a3, a4, a5, out_hbm, idx_v, rows_v, gsem0,
           gsem1, wsem0, wsem1):
    wid = lax.axis_index("s") * NC + lax.axis_index("c")
    tabs_r = (a0, a1, a2, a3, a4, a5)
    gsems = (gsem0, gsem1)
    wsems = (wsem0, wsem1)
    pltpu.sync_copy(idx_hbm.at[wid], idx_v)

    def fire(c):
      k = c // NCH
      pltpu.async_copy(
          tabs_r[k].at[idx_v.at[c]], rows_v.at[c % SLOTS], gsems[c % 2])

    def gwait(c):
      k = c // NCH
      pltpu.make_async_copy(
          tabs_r[k].at[idx_v.at[c]], rows_v.at[c % SLOTS],
          gsems[c % 2]).wait()

    def wstart(c):
      k, j = divmod(c, NCH)
      pltpu.async_copy(
          rows_v.at[c % SLOTS], out_hbm.at[k, wid, j], wsems[c % 2])

    def wwait(c):
      k, j = divmod(c, NCH)
      pltpu.make_async_copy(
          rows_v.at[c % SLOTS], out_hbm.at[k, wid, j],
          wsems[c % 2]).wait()

    fire(0)
    fire(1)
    for c in range(NCHUNK):
      gwait(c)
      if c >= 2:
        wwait(c - 2)
      wstart(c)
      if c + 2 < NCHUNK:
        fire(c + 2)
    for c in range(NCHUNK - 2, NCHUNK):
      wwait(c)

  return body(idxp, *tabs)


def _tc_project(g, half, w, bias):
  bm = 1024

  def body(g_ref, p_ref, w_ref, b_ref, o_ref):
    acc = jnp.broadcast_to(b_ref[0], (bm, H))
    for k in range(K):
      gk = g_ref[k]
      pk = p_ref[k][:, None]
      ek = gk[:, :D] * (1.0 - pk) + gk[:, D:] * pk
      acc = acc + jnp.dot(ek, w_ref[k], preferred_element_type=jnp.float32)
    o_ref[...] = acc

  return pl.pallas_call(
      body,
      grid=(B // bm,),
      in_specs=[
          pl.BlockSpec((K, bm, 2 * D), lambda i: (0, i, 0)),
          pl.BlockSpec((8, bm), lambda i: (0, i)),
          pl.BlockSpec((K, D, H), lambda i: (0, 0, 0)),
          pl.BlockSpec((1, H), lambda i: (0, 0)),
      ],
      out_specs=pl.BlockSpec((bm, H), lambda i: (i, 0)),
      out_shape=jax.ShapeDtypeStruct((B, H), jnp.float32),
  )(g, half, w, bias)


def kernel(batch_seq_cat, t_incbd, t_bldcnt, t_floorn, t_area, t_lon, t_lat,
           W, b):
  idx = batch_seq_cat[:, 1:7].astype(jnp.int32).T
  hbit = (idx >= M).astype(jnp.int32)
  idxp = (idx - hbit * M).reshape(K, NW, NCH, CH).transpose(1, 0, 2, 3)
  idxp = idxp.reshape(NW, NCHUNK, CH)
  half = jnp.pad(hbit.astype(jnp.float32), ((0, 2), (0, 0)))
  packed = _tc_pack(t_incbd.T, t_bldcnt.T, t_floorn.T, t_area.T, t_lon.T,
                    t_lat.T)
  g = _sc_gather(idxp, *packed)
  g = g.reshape(K, B, 2 * D)
  w = W.reshape(K, D, H)
  bias = b.reshape(1, H)
  return _tc_project(g, half, w, bias)


# speedup vs baseline: 3.3731x; 1.0016x over previous
import functools

import jax
import jax.numpy as jnp
from jax import lax
from jax.experimental import pallas as pl
from jax.experimental.pallas import tpu as pltpu
from jax.experimental.pallas import tpu_sc as plsc

B = 16384
V = 100000
D = 64
H = 128
K = 6
NC = 2
NS = 16
NW = NC * NS
BPW = B // NW
CH = 128
NCH = BPW // CH
NCHUNK = K * NCH
SLOTS = 4

TB = 3584
M = 50176
NTB = M // TB


def _tc_pack(*tabs_t):
  def body(*refs):
    ins, outs = refs[:2 * K], refs[2 * K:]
    for k in range(K):
      xc = jnp.concatenate([ins[2 * k][...], ins[2 * k + 1][...]], axis=0)
      outs[k][...] = xc.T

  in_specs = []
  for _ in range(K):
    in_specs.append(pl.BlockSpec((D, TB), lambda i: (0, i)))
    in_specs.append(pl.BlockSpec((D, TB), lambda i: (0, i + NTB)))
  return pl.pallas_call(
      body,
      grid=(NTB,),
      in_specs=in_specs,
      out_specs=[pl.BlockSpec((TB, 2 * D), lambda i: (i, 0))] * K,
      out_shape=[jax.ShapeDtypeStruct((M, 2 * D), jnp.float32)] * K,
  )(*[t for t in tabs_t for _ in range(2)])


def _sc_gather(idxp, *tabs):
  mesh = plsc.VectorSubcoreMesh(core_axis_name="c", subcore_axis_name="s")

  @functools.partial(
      pl.kernel,
      out_type=jax.ShapeDtypeStruct((K, NW, NCH, CH, 2 * D), jnp.float32),
      mesh=mesh,
      scratch_types=[
          pltpu.VMEM((NCHUNK, CH), jnp.int32),
          pltpu.VMEM((SLOTS, CH, 2 * D), jnp.float32),
          pltpu.SemaphoreType.DMA,
          pltpu.SemaphoreType.DMA,
          pltpu.SemaphoreType.DMA,
          pltpu.SemaphoreType.DMA,
      ],
  )
  def body(idx_hbm, a0, a1, a2, a3, a4, a5, out_hbm, idx_v, rows_v, gsem0,
           gsem1, wsem0, wsem1):
    wid = lax.axis_index("s") * NC + lax.axis_index("c")
    tabs_r = (a0, a1, a2, a3, a4, a5)
    gsems = (gsem0, gsem1)
    wsems = (wsem0, wsem1)
    pltpu.sync_copy(idx_hbm.at[wid], idx_v)

    def fire(c):
      k = c // NCH
      pltpu.async_copy(
          tabs_r[k].at[idx_v.at[c]], rows_v.at[c % SLOTS], gsems[c % 2])

    def gwait(c):
      k = c // NCH
      pltpu.make_async_copy(
          tabs_r[k].at[idx_v.at[c]], rows_v.at[c % SLOTS],
          gsems[c % 2]).wait()

    def wstart(c):
      k, j = divmod(c, NCH)
      pltpu.async_copy(
          rows_v.at[c % SLOTS], out_hbm.at[k, wid, j], wsems[c % 2])

    def wwait(c):
      k, j = divmod(c, NCH)
      pltpu.make_async_copy(
          rows_v.at[c % SLOTS], out_hbm.at[k, wid, j],
          wsems[c % 2]).wait()

    fire(0)
    fire(1)
    for c in range(NCHUNK):
      gwait(c)
      if c >= 2:
        wwait(c - 2)
      wstart(c)
      if c + 2 < NCHUNK:
        fire(c + 2)
    for c in range(NCHUNK - 2, NCHUNK):
      wwait(c)

  return body(idxp, *tabs)


def _tc_project(g, half, w, bias):
  bm = 1024

  def body(g_ref, p_ref, w_ref, b_ref, o_ref):
    acc = jnp.broadcast_to(b_ref[0], (bm, H))
    for k in range(K):
      gk = g_ref[k]
      pk = p_ref[k][:, None]
      ek = gk[:, :D] * (1.0 - pk) + gk[:, D:] * pk
      acc = acc + jnp.dot(ek, w_ref[k], preferred_element_type=jnp.float32)
    o_ref[...] = acc

  return pl.pallas_call(
      body,
      grid=(B // bm,),
      in_specs=[
          pl.BlockSpec((K, bm, 2 * D), lambda i: (0, i, 0)),
          pl.BlockSpec((8, bm), lambda i: (0, i)),
          pl.BlockSpec((K, D, H), lambda i: (0, 0, 0)),
          pl.BlockSpec((1, H), lambda i: (0, 0)),
      ],
      out_specs=pl.BlockSpec((bm, H), lambda i: (i, 0)),
      out_shape=jax.ShapeDtypeStruct((B, H), jnp.float32),
  )(g, half, w, bias)


def kernel(batch_seq_cat, t_incbd, t_bldcnt, t_floorn, t_area, t_lon, t_lat,
           W, b):
  idx = batch_seq_cat[:, 1:7].astype(jnp.int32).T
  hbit = (idx >= M).astype(jnp.int32)
  idxp = (idx - hbit * M).reshape(K, NW, NCH, CH).transpose(1, 0, 2, 3)
  idxp = idxp.reshape(NW, NCHUNK, CH)
  half = jnp.pad(hbit.astype(jnp.float32), ((0, 2), (0, 0)))
  packed = _tc_pack(t_incbd.T, t_bldcnt.T, t_floorn.T, t_area.T, t_lon.T,
                    t_lat.T)
  g = _sc_gather(idxp, *packed)
  g = g.reshape(K, B, 2 * D)
  w = W.reshape(K, D, H)
  bias = b.reshape(1, H)
  return _tc_project(g, half, w, bias)


# speedup vs baseline: 3.4426x; 1.0206x over previous
import functools

import jax
import jax.numpy as jnp
from jax import lax
from jax.experimental import pallas as pl
from jax.experimental.pallas import tpu as pltpu
from jax.experimental.pallas import tpu_sc as plsc

B = 16384
V = 100000
D = 64
H = 128
K = 6
NC = 2
NS = 16
NW = NC * NS
BPW = B // NW
CH = 128
NCH = BPW // CH
NCHUNK = K * NCH
SLOTS = 4

TB = 1792
M = 25088
NTB = M // TB


def _tc_pack(*tabs_t):
  def rne16(x):
    xi = jax.lax.bitcast_convert_type(x, jnp.int32)
    bias = 0x7FFF + ((xi >> 16) & 1)
    return xi + bias

  def body(*refs):
    ins, outs = refs[:4 * K], refs[4 * K:]
    for k in range(K):
      xc = jnp.concatenate([ins[4 * k + q][...] for q in range(4)], axis=0)
      y = xc.T
      lo = jax.lax.shift_right_logical(rne16(y[:, :2 * D]), 16)
      hi = rne16(y[:, 2 * D:]) & jnp.int32(-65536)
      outs[k][...] = jax.lax.bitcast_convert_type(lo | hi, jnp.float32)

  in_specs = []
  for _ in range(K):
    in_specs.append(pl.BlockSpec((D, TB), lambda i: (0, i)))
    in_specs.append(pl.BlockSpec((D, TB), lambda i: (0, i + NTB)))
    in_specs.append(pl.BlockSpec((D, TB), lambda i: (0, i + 2 * NTB)))
    in_specs.append(pl.BlockSpec((D, TB), lambda i: (0, i + 3 * NTB)))
  return pl.pallas_call(
      body,
      grid=(NTB,),
      in_specs=in_specs,
      out_specs=[pl.BlockSpec((TB, 2 * D), lambda i: (i, 0))] * K,
      out_shape=[jax.ShapeDtypeStruct((M, 2 * D), jnp.float32)] * K,
  )(*[t for t in tabs_t for _ in range(4)])


def _sc_gather(idxp, *tabs):
  mesh = plsc.VectorSubcoreMesh(core_axis_name="c", subcore_axis_name="s")

  @functools.partial(
      pl.kernel,
      out_type=jax.ShapeDtypeStruct((K, NW, NCH, CH, 2 * D), jnp.float32),
      mesh=mesh,
      scratch_types=[
          pltpu.VMEM((NCHUNK, CH), jnp.int32),
          pltpu.VMEM((SLOTS, CH, 2 * D), jnp.float32),
          pltpu.SemaphoreType.DMA,
          pltpu.SemaphoreType.DMA,
          pltpu.SemaphoreType.DMA,
          pltpu.SemaphoreType.DMA,
      ],
  )
  def body(idx_hbm, a0, a1, a2, a3, a4, a5, out_hbm, idx_v, rows_v, gsem0,
           gsem1, wsem0, wsem1):
    wid = lax.axis_index("s") * NC + lax.axis_index("c")
    tabs_r = (a0, a1, a2, a3, a4, a5)
    gsems = (gsem0, gsem1)
    wsems = (wsem0, wsem1)
    pltpu.sync_copy(idx_hbm.at[wid], idx_v)

    def fire(c):
      k = c // NCH
      pltpu.async_copy(
          tabs_r[k].at[idx_v.at[c]], rows_v.at[c % SLOTS], gsems[c % 2])

    def gwait(c):
      k = c // NCH
      pltpu.make_async_copy(
          tabs_r[k].at[idx_v.at[c]], rows_v.at[c % SLOTS],
          gsems[c % 2]).wait()

    def wstart(c):
      k, j = divmod(c, NCH)
      pltpu.async_copy(
          rows_v.at[c % SLOTS], out_hbm.at[k, wid, j], wsems[c % 2])

    def wwait(c):
      k, j = divmod(c, NCH)
      pltpu.make_async_copy(
          rows_v.at[c % SLOTS], out_hbm.at[k, wid, j],
          wsems[c % 2]).wait()

    fire(0)
    fire(1)
    for c in range(NCHUNK):
      gwait(c)
      if c >= 2:
        wwait(c - 2)
      wstart(c)
      if c + 2 < NCHUNK:
        fire(c + 2)
    for c in range(NCHUNK - 2, NCHUNK):
      wwait(c)

  return body(idxp, *tabs)


def _tc_project(g, half, w, bias):
  bm = 1024

  def body(g_ref, p_ref, w_ref, b_ref, o_ref):
    acc = jnp.broadcast_to(b_ref[0], (bm, H))
    for k in range(K):
      u = jax.lax.bitcast_convert_type(g_ref[k], jnp.int32)
      flo = jax.lax.bitcast_convert_type(
          jax.lax.shift_left(u, 16), jnp.float32)
      fhi = jax.lax.bitcast_convert_type(
          u & jnp.int32(-65536), jnp.float32)
      qk = p_ref[k][:, None]
      ek = (flo[:, :D] * (qk == 0.).astype(jnp.float32)
            + flo[:, D:] * (qk == 1.).astype(jnp.float32)
            + fhi[:, :D] * (qk == 2.).astype(jnp.float32)
            + fhi[:, D:] * (qk == 3.).astype(jnp.float32))
      acc = acc + jnp.dot(ek, w_ref[k], preferred_element_type=jnp.float32)
    o_ref[...] = acc

  return pl.pallas_call(
      body,
      grid=(B // bm,),
      in_specs=[
          pl.BlockSpec((K, bm, 2 * D), lambda i: (0, i, 0)),
          pl.BlockSpec((8, bm), lambda i: (0, i)),
          pl.BlockSpec((K, D, H), lambda i: (0, 0, 0)),
          pl.BlockSpec((1, H), lambda i: (0, 0)),
      ],
      out_specs=pl.BlockSpec((bm, H), lambda i: (i, 0)),
      out_shape=jax.ShapeDtypeStruct((B, H), jnp.float32),
  )(g, half, w, bias)


def kernel(batch_seq_cat, t_incbd, t_bldcnt, t_floorn, t_area, t_lon, t_lat,
           W, b):
  idx = batch_seq_cat[:, 1:7].astype(jnp.int32).T
  q = idx // M
  idxp = (idx - q * M).reshape(K, NW, NCH, CH).transpose(1, 0, 2, 3)
  idxp = idxp.reshape(NW, NCHUNK, CH)
  half = jnp.pad(q.astype(jnp.float32), ((0, 2), (0, 0)))
  packed = _tc_pack(t_incbd.T, t_bldcnt.T, t_floorn.T, t_area.T, t_lon.T,
                    t_lat.T)
  g = _sc_gather(idxp, *packed)
  g = g.reshape(K, B, 2 * D)
  w = W.reshape(K, D, H)
  bias = b.reshape(1, H)
  return _tc_project(g, half, w, bias)
